# baseline (speedup 1.0000x reference)
import functools

import jax
import jax.numpy as jnp
from jax import lax
from jax.experimental import pallas as pl
from jax.experimental.pallas import tpu as pltpu
from jax.experimental.pallas import tpu_sc as plsc

_B, _M, _H, _W = 16, 8, 512, 512
_P = 16
_G = _H // _P
_N = _G * _G
_D = 384
_NC = 4
_BB = 4

_NCORES = 2
_NSUB = 16
_NWORK = _NCORES * _NSUB
_PAIRS = _B * _M
_PPW = _PAIRS // _NWORK


def _sc_pool(mask_hbm, out_hbm, inbuf, poolbuf):
    wid = lax.axis_index("s") * _NCORES + lax.axis_index("c")
    for t in range(_PPW):
        p = wid * _PPW + t
        pltpu.sync_copy(mask_hbm.at[p, :, 0, :], inbuf)
        for g in range(_G):
            for h in range(2):
                col = 256 * h + lax.iota(jnp.int32, 16) * _P
                row = jnp.full((16,), g, jnp.int32)
                vec = plsc.load_gather(inbuf, [row, col])
                poolbuf[t * _G + g, pl.ds(16 * h, 16)] = vec
    pltpu.sync_copy(poolbuf, out_hbm.at[pl.ds(wid * _PPW * _G, _PPW * _G), :])


def _tc_body(cat_ref, pool_ref, emb_ref, oute_ref, outf_ref):
    j = pl.program_id(0)

    w4_t = []
    spc_t = []
    for t in range(_BB):
        pooledr = pool_ref[0, t]

        wc = []
        for c in range(_NC):
            acc = jnp.zeros((_G, _G), jnp.float32)
            for m in range(_M):
                ind = jnp.where(cat_ref[0, t, m] == c, 1.0, 0.0)
                acc = acc + pooledr[m] * ind
            wc.append(acc)
        w4 = jnp.stack(wc, axis=0)
        w4_t.append(w4)

        spc_g = lax.dot_general(
            w4, emb_ref[t],
            dimension_numbers=(((2,), (1,)), ((1,), (0,))),
            preferred_element_type=jnp.float32)
        spc_t.append(jnp.sum(spc_g, axis=0))

    @pl.when(j == 0)
    def _init():
        oute_ref[...] = jnp.zeros_like(oute_ref)
        outf_ref[...] = jnp.zeros_like(outf_ref)

    outf_ref[...] = outf_ref[...] + sum(w4_t)
    oute_ref[...] = oute_ref[...] + sum(spc_t)

    @pl.when(j == _B // _BB - 1)
    def _finish():
        cnt = jnp.sum(outf_ref[...], axis=(1, 2))
        mean = oute_ref[...] / jnp.maximum(cnt, 1.0)[:, None]
        nrm = jnp.sqrt(jnp.sum(mean * mean, axis=1, keepdims=True))
        oute_ref[...] = mean / jnp.maximum(nrm, 1e-12)


def kernel(embeddings, masks, category_ids):
    masks_v = masks.reshape(_PAIRS, _G, _P, _W)
    emb_r = embeddings.reshape(_B, _G, _G, _D)
    cat_r = category_ids.reshape(_B // _BB, _BB, _M)

    mesh = plsc.VectorSubcoreMesh(core_axis_name="c", subcore_axis_name="s")
    pooled = pl.kernel(
        _sc_pool,
        mesh=mesh,
        compiler_params=pltpu.CompilerParams(use_tc_tiling_on_sc=False, needs_layout_passes=False),
        out_type=jax.ShapeDtypeStruct((_PAIRS * _G, _G), jnp.float32),
        scratch_types=[
            pltpu.VMEM((_G, _W), jnp.float32),
            pltpu.VMEM((_PPW * _G, _G), jnp.float32),
        ],
    )(masks_v)
    pooled_v = pooled.reshape(_B // _BB, _BB, _M, _G, _G)

    out_emb, out_flat = pl.pallas_call(
        _tc_body,
        grid=(_B // _BB,),
        in_specs=[
            pl.BlockSpec((1, _BB, _M), lambda j: (j, 0, 0),
                         memory_space=pltpu.SMEM),
            pl.BlockSpec((1, _BB, _M, _G, _G), lambda j: (j, 0, 0, 0, 0)),
            pl.BlockSpec((_BB, _G, _G, _D), lambda j: (j, 0, 0, 0)),
        ],
        out_specs=[
            pl.BlockSpec((_NC, _D), lambda j: (0, 0)),
            pl.BlockSpec((_NC, _G, _G), lambda j: (0, 0, 0)),
        ],
        out_shape=[
            jax.ShapeDtypeStruct((_NC, _D), jnp.float32),
            jax.ShapeDtypeStruct((_NC, _G, _G), jnp.float32),
        ],
    )(cat_r, pooled_v, emb_r)

    return out_emb, out_flat.reshape(_NC, _N)
